# Initial kernel scaffold; baseline (speedup 1.0000x reference)
import jax
import jax.numpy as jnp
from jax.experimental import pallas as pl


def kernel(x, edge_index, batch, W, bias, W2, b2):
    raise NotImplementedError("write your pallas kernel here")


# speedup vs baseline: 50.6368x; 50.6368x over previous
import functools

import jax
import jax.numpy as jnp
from jax import lax
from jax.experimental import pallas as pl
from jax.experimental.pallas import tpu as pltpu
from jax.experimental.pallas import tpu_sc as plsc

N = 10000
E = 320000
D = 128
G = 256
KSEL = 2
NPAD = 10240
NROW = NPAD // 16
NT = 32
ET = E // NT
CH = 100
NCH = ET // CH
EC = E // 16
BIGI = 1 << 20

_mesh = plsc.VectorSubcoreMesh(core_axis_name="c", subcore_axis_name="s")


def _rsqrt16(v):
    i = plsc.bitcast(v, jnp.int32)
    i = jnp.int32(0x5F3759DF) - lax.shift_right_logical(i, 1)
    g = plsc.bitcast(i, jnp.float32)
    for _ in range(3):
        g = g * (jnp.float32(1.5) - jnp.float32(0.5) * v * g * g)
    return g


def _extract_i32(vec, lane):
    io = lax.iota(jnp.int32, 16)
    return jnp.max(jnp.where(io == lane, vec, jnp.int32(-1)))


@functools.partial(
    pl.kernel,
    out_type=(
        jax.ShapeDtypeStruct((80, 128), jnp.float32),
        jax.ShapeDtypeStruct((2, 128), jnp.int32),
        jax.ShapeDtypeStruct((2, 128), jnp.int32),
    ),
    mesh=_mesh,
    compiler_params=pltpu.CompilerParams(needs_layout_passes=False),
    scratch_types=[
        pltpu.VMEM((EC,), jnp.int32),
        pltpu.VMEM((80, 128), jnp.float32),
        pltpu.VMEM((1, 80), jnp.int32),
        pltpu.VMEM((640,), jnp.int32),
        pltpu.VMEM((16, 128), jnp.float32),
        pltpu.VMEM((1, 16), jnp.int32),
        pltpu.VMEM((2, 128), jnp.int32),
        pltpu.VMEM((2, 128), jnp.int32),
        pltpu.VMEM((8, 128), jnp.float32),
        pltpu.MemorySpace.VMEM_SHARED((80, 128), jnp.float32),
        pltpu.MemorySpace.VMEM_SHARED((16, 128), jnp.float32),
    ],
)
def _sc_stats(col_hbm, batch_hbm, dinv_hbm, offs_hbm, cnts_hbm,
              colv, deg2d, idxd, bv, c2d, idxc, obuf, cbuf, dbuf,
              sh_deg, sh_cnt):
    c = lax.axis_index("c")
    s = lax.axis_index("s")
    io = lax.iota(jnp.int32, 16)
    zf = jnp.zeros((16,), jnp.float32)
    ones = jnp.ones((16,), jnp.float32)

    @pl.when(c == 0)
    def _deg_path():
        for r in range(80):
            for q in range(8):
                deg2d[r, pl.ds(q * 16, 16)] = zf
        for k in range(5):
            idxd[0, pl.ds(k * 16, 16)] = jnp.int32(k * 16) + io

        @pl.when(s == 0)
        def _zero_shared_deg():
            pltpu.sync_copy(deg2d, sh_deg)

        plsc.subcore_barrier()

        pltpu.sync_copy(col_hbm.at[pl.ds(s * EC, EC)], colv)

        def body(i, _):
            cv = colv[pl.ds(i * 16, 16)]
            r = lax.shift_right_logical(cv, 7)
            l = jnp.bitwise_and(cv, jnp.int32(127))
            plsc.addupdate_scatter(deg2d, [r, l], ones)
            return 0

        lax.fori_loop(0, EC // 16, body, 0)

        pltpu.sync_copy(deg2d, sh_deg.at[idxd.at[0]], add=True)
        plsc.subcore_barrier()

        @pl.when(s < 10)
        def _dinv():
            pltpu.sync_copy(sh_deg.at[pl.ds(s * 8, 8)], dbuf)
            for r in range(8):
                for q in range(8):
                    dbuf[r, pl.ds(q * 16, 16)] = _rsqrt16(
                        dbuf[r, pl.ds(q * 16, 16)] + jnp.float32(1.0))
            pltpu.sync_copy(dbuf, dinv_hbm.at[pl.ds(s * 8, 8)])

    @pl.when(c == 1)
    def _cnt_path():
        for r in range(16):
            for q in range(8):
                c2d[r, pl.ds(q * 16, 16)] = zf
        idxc[0, :] = io

        @pl.when(s == 0)
        def _zero_shared_cnt():
            pltpu.sync_copy(c2d, sh_cnt)

        plsc.subcore_barrier()

        pltpu.sync_copy(batch_hbm.at[pl.ds(s * 640, 640)], bv)

        def body(i, _):
            gv = bv[pl.ds(i * 16, 16)]
            r = lax.shift_right_logical(gv, 7)
            l = jnp.bitwise_and(gv, jnp.int32(127))
            plsc.addupdate_scatter(c2d, [r, l], ones)
            return 0

        lax.fori_loop(0, 40, body, 0)
        pltpu.sync_copy(c2d, sh_cnt.at[idxc.at[0]], add=True)
        plsc.subcore_barrier()

        @pl.when(s == 0)
        def _offsets():
            pltpu.sync_copy(sh_cnt, c2d)
            carry = jnp.float32(0.0)
            for r in range(2):
                for q in range(8):
                    v = c2d[r, pl.ds(q * 16, 16)]
                    incl = plsc.cumsum(v)
                    obuf[r, pl.ds(q * 16, 16)] = (incl - v + carry).astype(
                        jnp.int32)
                    cbuf[r, pl.ds(q * 16, 16)] = v.astype(jnp.int32)
                    carry = carry + jnp.sum(v)
            pltpu.sync_copy(obuf, offs_hbm)
            pltpu.sync_copy(cbuf, cnts_hbm)


@functools.partial(
    pl.kernel,
    out_type=jax.ShapeDtypeStruct((2, N, D), jnp.float32),
    mesh=_mesh,
    compiler_params=pltpu.CompilerParams(needs_layout_passes=False),
    scratch_types=[
        pltpu.VMEM((NCH // 2, CH), jnp.int32),
        pltpu.VMEM((NCH // 2, CH), jnp.int32),
        pltpu.VMEM((CH, D), jnp.float32),
        pltpu.VMEM((CH, D), jnp.float32),
        pltpu.VMEM((8, D), jnp.float32),
        pltpu.MemorySpace.VMEM_SHARED((N, D), jnp.float32),
        pltpu.SemaphoreType.DMA,
        pltpu.SemaphoreType.DMA,
    ],
)
def _sc_scatter(row_hbm, col_hbm, y_hbm, acc_hbm,
                rowv, colv, gb0, gb1, zb, acc_sh, sem0, sem1):
    c = lax.axis_index("c")
    s = lax.axis_index("s")
    wid = c * 16 + s
    zf = jnp.zeros((16,), jnp.float32)

    for r in range(8):
        for q in range(D // 16):
            zb[r, pl.ds(q * 16, 16)] = zf
    for i in range(78):
        pltpu.sync_copy(zb, acc_sh.at[pl.ds(s * 624 + i * 8, 8)])

    @pl.when(s == 15)
    def _zero_tail():
        pltpu.sync_copy(zb.at[pl.ds(0, 16)], acc_sh.at[pl.ds(9984, 16)])

    plsc.subcore_barrier()

    HC = NCH // 2
    for half in range(2):
        pltpu.sync_copy(row_hbm.at[wid].at[half], rowv)
        pltpu.sync_copy(col_hbm.at[wid].at[half], colv)
        pltpu.async_copy(y_hbm.at[rowv.at[0]], gb0, sem0)
        pltpu.async_copy(y_hbm.at[rowv.at[1]], gb1, sem1)

        def body(i, _):
            k0 = 2 * i
            pltpu.make_async_copy(y_hbm.at[rowv.at[0]], gb0, sem0).wait()
            pltpu.sync_copy(gb0, acc_sh.at[colv.at[k0]], add=True)

            @pl.when(i < HC // 2 - 1)
            def _issue0():
                pltpu.async_copy(y_hbm.at[rowv.at[k0 + 2]], gb0, sem0)

            pltpu.make_async_copy(y_hbm.at[rowv.at[0]], gb1, sem1).wait()
            pltpu.sync_copy(gb1, acc_sh.at[colv.at[k0 + 1]], add=True)

            @pl.when(i < HC // 2 - 1)
            def _issue1():
                pltpu.async_copy(y_hbm.at[rowv.at[k0 + 3]], gb1, sem1)

            return 0

        lax.fori_loop(0, HC // 2, body, 0)
    plsc.subcore_barrier()

    pltpu.sync_copy(acc_sh.at[pl.ds(s * 624, 624)],
                    acc_hbm.at[c].at[pl.ds(s * 624, 624)])

    @pl.when(s == 15)
    def _pub_tail():
        pltpu.sync_copy(acc_sh.at[pl.ds(9984, 16)],
                        acc_hbm.at[c].at[pl.ds(9984, 16)])


@functools.partial(
    pl.kernel,
    out_type=jax.ShapeDtypeStruct((2 * G, D), jnp.float32),
    mesh=_mesh,
    compiler_params=pltpu.CompilerParams(needs_layout_passes=False),
    scratch_types=[
        pltpu.VMEM((N,), jnp.float32),
        pltpu.VMEM((G,), jnp.int32),
        pltpu.VMEM((G,), jnp.int32),
        pltpu.VMEM((1, 16), jnp.int32),
        pltpu.VMEM((16, D), jnp.float32),
        pltpu.SemaphoreType.DMA,
    ],
)
def _sc_topk(s_hbm, offs_hbm, cnts_hbm, h_hbm, sel_hbm,
             sv, ov, cv, idxb, rows, sem):
    c = lax.axis_index("c")
    s = lax.axis_index("s")
    w = c * 16 + s
    io = lax.iota(jnp.int32, 16)
    zf = jnp.zeros((16,), jnp.float32)

    pltpu.sync_copy(s_hbm, sv)
    pltpu.sync_copy(offs_hbm, ov)
    pltpu.sync_copy(cnts_hbm, cv)

    idxvec = jnp.zeros((16,), jnp.int32)
    flags = []
    for j in range(8):
        gi = 8 * w + j
        blk = jnp.bitwise_and(gi, jnp.int32(~15))
        lane = jnp.bitwise_and(gi, jnp.int32(15))
        off = _extract_i32(ov[pl.ds(blk, 16)], lane)
        cnt = _extract_i32(cv[pl.ds(blk, 16)], lane)
        nchunk = lax.shift_right_logical(cnt + jnp.int32(15), 1 + 3)

        def body(t, carry):
            b1v, b1i, b2v, b2i = carry
            idx = off + t * 16 + io
            valid = idx < off + cnt
            v = plsc.load_gather(sv, [jnp.where(valid, idx, 0)])
            v = jnp.where(valid, v, jnp.float32(-1.0))
            ii = jnp.where(valid, idx, BIGI)
            isb1 = (v > b1v) | ((v == b1v) & (ii < b1i))
            isb2 = jnp.logical_not(isb1) & ((v > b2v) | ((v == b2v) & (ii < b2i)))
            nb2v = jnp.where(isb1, b1v, jnp.where(isb2, v, b2v))
            nb2i = jnp.where(isb1, b1i, jnp.where(isb2, ii, b2i))
            return (jnp.where(isb1, v, b1v), jnp.where(isb1, ii, b1i),
                    nb2v, nb2i)

        neg = jnp.full((16,), -1.0, jnp.float32)
        big = jnp.full((16,), 1 << 20, jnp.int32)
        b1v, b1i, b2v, b2i = lax.fori_loop(
            0, nchunk, body, (neg, big, neg, big))

        m1 = jnp.max(b1v)
        i1 = jnp.minimum(jnp.min(jnp.where(b1v == m1, b1i, BIGI)),
                         jnp.min(jnp.where(b2v == m1, b2i, BIGI)))
        x1 = jnp.where((b1v == m1) & (b1i == i1), jnp.float32(-2.0), b1v)
        x2 = jnp.where((b2v == m1) & (b2i == i1), jnp.float32(-2.0), b2v)
        m2 = jnp.maximum(jnp.max(x1), jnp.max(x2))
        i2 = jnp.minimum(jnp.min(jnp.where(x1 == m2, b1i, BIGI)),
                         jnp.min(jnp.where(x2 == m2, b2i, BIGI)))
        i1 = jnp.where(cnt >= 1, i1, 0)
        i2 = jnp.where(cnt >= 2, i2, 0)
        idxvec = jnp.where(io == 2 * j, i1, idxvec)
        idxvec = jnp.where(io == 2 * j + 1, i2, idxvec)
        flags.append(cnt)

    idxb[0, :] = idxvec
    pltpu.async_copy(h_hbm.at[idxb.at[0]], rows, sem).wait()

    for j in range(8):
        @pl.when(flags[j] < 1)
        def _z1():
            for q in range(D // 16):
                rows[2 * j, pl.ds(q * 16, 16)] = zf

        @pl.when(flags[j] < 2)
        def _z2():
            for q in range(D // 16):
                rows[2 * j + 1, pl.ds(q * 16, 16)] = zf

    pltpu.sync_copy(rows, sel_hbm.at[pl.ds(w * 16, 16)])


def _tc_y(x, W, dinv2d):
    def body(x_ref, w_ref, d_ref, y_ref):
        y_ref[...] = jnp.dot(x_ref[...], w_ref[...],
                             preferred_element_type=jnp.float32) * d_ref[...]

    return pl.pallas_call(
        body,
        grid=(10,),
        in_specs=[
            pl.BlockSpec((N // 10, D), lambda i: (i, 0)),
            pl.BlockSpec((D, D), lambda i: (0, 0)),
            pl.BlockSpec((N // 10, 1), lambda i: (i, 0)),
        ],
        out_specs=pl.BlockSpec((N // 10, D), lambda i: (i, 0)),
        out_shape=jax.ShapeDtypeStruct((N, D), jnp.float32),
    )(x, W, dinv2d)


def _tc_finalize(a0, a1, y, dinv2d, bias2d):
    def body(a0_ref, a1_ref, y_ref, d_ref, b_ref, h_ref, s_ref):
        t = (a0_ref[...] + a1_ref[...] + y_ref[...]) * d_ref[...] + b_ref[...]
        h = jnp.maximum(t, 0.0)
        h_ref[...] = h
        s_ref[...] = h[:, D - 1:D]

    return pl.pallas_call(
        body,
        grid=(10,),
        in_specs=[
            pl.BlockSpec((N // 10, D), lambda i: (i, 0)),
            pl.BlockSpec((N // 10, D), lambda i: (i, 0)),
            pl.BlockSpec((N // 10, D), lambda i: (i, 0)),
            pl.BlockSpec((N // 10, 1), lambda i: (i, 0)),
            pl.BlockSpec((1, D), lambda i: (0, 0)),
        ],
        out_specs=[
            pl.BlockSpec((N // 10, D), lambda i: (i, 0)),
            pl.BlockSpec((N // 10, 1), lambda i: (i, 0)),
        ],
        out_shape=[
            jax.ShapeDtypeStruct((N, D), jnp.float32),
            jax.ShapeDtypeStruct((N, 1), jnp.float32),
        ],
    )(a0, a1, y, dinv2d, bias2d)


def _tc_classify(flat, W2, b2_2d):
    def body(f_ref, w_ref, b_ref, o_ref):
        o_ref[...] = jnp.dot(f_ref[...], w_ref[...],
                             preferred_element_type=jnp.float32) + b_ref[...]

    return pl.pallas_call(
        body,
        in_specs=[
            pl.BlockSpec((G, 2 * D), lambda: (0, 0)),
            pl.BlockSpec((2 * D, D), lambda: (0, 0)),
            pl.BlockSpec((1, D), lambda: (0, 0)),
        ],
        out_specs=pl.BlockSpec((G, D), lambda: (0, 0)),
        out_shape=jax.ShapeDtypeStruct((G, D), jnp.float32),
    )(flat, W2, b2_2d)


def kernel(x, edge_index, batch, W, bias, W2, b2):
    row = edge_index[0].reshape(NT, 2, NCH // 2, CH)
    col_flat = edge_index[1]
    col = col_flat.reshape(NT, 2, NCH // 2, CH)
    batch_pad = jnp.concatenate(
        [batch, jnp.full((NPAD - N,), G, jnp.int32)])

    dinv_t, offs_t, cnts_t = _sc_stats(col_flat, batch_pad)
    dinv2d = dinv_t.reshape(NPAD, 1)[:N]

    y = _tc_y(x, W, dinv2d)
    accp = _sc_scatter(row, col, y)
    h, s2d = _tc_finalize(accp[0], accp[1], y, dinv2d, bias.reshape(1, D))
    sel = _sc_topk(s2d.reshape(N), offs_t.reshape(G), cnts_t.reshape(G), h)
    out = _tc_classify(sel.reshape(G, 2 * D), W2, b2.reshape(1, D))
    return out
